# baseline (speedup 1.0000x reference)
import functools

import jax
import jax.numpy as jnp
from jax import lax
from jax.experimental import pallas as pl
import jax.experimental.pallas.tpu as pltpu
from jax.experimental.pallas import tpu_sc as plsc

_B, _H, _W = 4, 8, 128
_C = 16
_K = 1024
_D = 128
_TK = 15
_PAD = 7
_EPS = 1e-5
_V = _B * _C * _H
_KB = 256
_STEPS = _K // _KB


def _dist_kernel(ze_ref, eblk_ref, lat_ref, dists_ref, best_ref, bidx_ref):
    step = pl.program_id(0)

    @pl.when(step == 0)
    def _init():
        best_ref[...] = jnp.full((_V, 1), jnp.inf, jnp.float32)
        bidx_ref[...] = jnp.zeros((_V, 1), jnp.int32)

    v = ze_ref[...].reshape(_V, _D)
    for j in range(_KB):
        diff = v - eblk_ref[j:j + 1, :]
        dcol = jnp.sum(diff * diff, axis=1, keepdims=True)
        dists_ref[:, j:j + 1] = dcol

    acc = dists_ref[...]
    lane_b = jax.lax.broadcasted_iota(jnp.int32, (_V, _KB), 1)
    m = jnp.min(acc, axis=1, keepdims=True)
    jblk = jnp.min(jnp.where(acc == m, lane_b, _KB), axis=1, keepdims=True)
    upd = m < best_ref[...]
    best_ref[...] = jnp.where(upd, m, best_ref[...])
    bidx_ref[...] = jnp.where(upd, step * _KB + jblk, bidx_ref[...])

    @pl.when(step == _STEPS - 1)
    def _fin():
        lat_ref[...] = bidx_ref[...]


def _dec_kernel(zq_ref, wcol_ref, xt_ref, pad_ref):
    zq = zq_ref[...]
    pad_ref[...] = jnp.zeros((_V, 2 * _W), jnp.float32)
    pad_ref[:, 0:_W] = zq
    wide = pad_ref[...]
    dec = jnp.zeros((_V, _W), jnp.float32)
    for t in range(_TK):
        shifted = pltpu.roll(wide, (_PAD - t) % (2 * _W), axis=1)[:, 0:_W]
        dec = dec + shifted * wcol_ref[t]
    for b in range(_B):
        blk = jnp.zeros((_H, _W), jnp.float32)
        for c in range(_C):
            blk = blk + dec[b * _C * _H + c * _H:b * _C * _H + c * _H + _H, :]
        xt_ref[b * _H:(b + 1) * _H, :] = jnp.tanh(blk)


_SC_INFO = plsc.get_sparse_core_info()
_NW = _SC_INFO.num_cores * _SC_INFO.num_subcores
_BPW = _V // _NW


def _make_sc_gather():
    mesh = plsc.VectorSubcoreMesh(core_axis_name="c", subcore_axis_name="s")

    @functools.partial(
        pl.kernel, mesh=mesh,
        out_type=jax.ShapeDtypeStruct((_V, _D), jnp.float32),
        scratch_types=[
            pltpu.VMEM((_BPW,), jnp.int32),
            pltpu.VMEM((_BPW, _D), jnp.float32),
            pltpu.SemaphoreType.DMA,
        ],
    )
    def k(table_hbm, idx_hbm, out_hbm, idx_v, rows_v, sem):
        wid = lax.axis_index("s") * _SC_INFO.num_cores + lax.axis_index("c")
        base = wid * _BPW
        pltpu.sync_copy(idx_hbm.at[pl.ds(base, _BPW)], idx_v)
        pltpu.async_copy(table_hbm.at[idx_v], rows_v, sem).wait()
        pltpu.sync_copy(rows_v, out_hbm.at[pl.ds(base, _BPW)])

    return k


_sc_gather = _make_sc_gather()


def kernel(x, w_enc, bn_gamma, bn_beta, emb, w_dec):
    z = jax.lax.conv_general_dilated(
        x, w_enc, window_strides=(1, 1), padding=((0, 0), (_PAD, _PAD)),
        dimension_numbers=("NCHW", "OIHW", "NCHW"))
    mean = jnp.mean(z, axis=(0, 2, 3), keepdims=True)
    var = jnp.var(z, axis=(0, 2, 3), keepdims=True)
    z_e_x = (z - mean) / jnp.sqrt(var + _EPS)
    z_e_x = z_e_x * bn_gamma.reshape(1, -1, 1, 1) + bn_beta.reshape(1, -1, 1, 1)

    lat = pl.pallas_call(
        _dist_kernel,
        grid=(_STEPS,),
        in_specs=[
            pl.BlockSpec((_B, _C, _H, _W), lambda s: (0, 0, 0, 0)),
            pl.BlockSpec((_KB, _D), lambda s: (s, 0)),
        ],
        out_specs=pl.BlockSpec((_V, 1), lambda s: (0, 0)),
        out_shape=jax.ShapeDtypeStruct((_V, 1), jnp.int32),
        scratch_shapes=[
            pltpu.VMEM((_V, _KB), jnp.float32),
            pltpu.VMEM((_V, 1), jnp.float32),
            pltpu.VMEM((_V, 1), jnp.int32),
        ],
    )(z_e_x, emb)

    zq = _sc_gather(emb, lat.reshape(_V))

    wt = w_dec[:, 0, 0, ::-1]
    c_idx = (jnp.arange(_V) // _H) % _C
    wcols = wt[c_idx].T[:, :, None]

    xt = pl.pallas_call(
        _dec_kernel,
        in_specs=[
            pl.BlockSpec((_V, _D), lambda: (0, 0)),
            pl.BlockSpec((_TK, _V, 1), lambda: (0, 0, 0)),
        ],
        out_specs=pl.BlockSpec((_B * _H, _W), lambda: (0, 0)),
        out_shape=jax.ShapeDtypeStruct((_B * _H, _W), jnp.float32),
        scratch_shapes=[pltpu.VMEM((_V, 2 * _W), jnp.float32)],
    )(zq, wcols)

    x_tilde = xt.reshape(_B, 1, _H, _W)
    z_q_x = zq.reshape(_B, _C, _H, _D)
    return (x_tilde, z_e_x, z_q_x)


# speedup vs baseline: 1.1964x; 1.1964x over previous
import jax
import jax.numpy as jnp
from jax.experimental import pallas as pl
import jax.experimental.pallas.tpu as pltpu

_B, _H, _W = 4, 8, 128
_C = 16
_K = 1024
_D = 128
_TK = 15
_PAD = 7
_EPS = 1e-5
_V = _B * _C * _H
_KB = 256
_STEPS = _K // _KB


def _vq_kernel(ze_ref, eblk_ref, emb_ref, wcol_ref, xt_ref, zq_ref,
               best_ref, bidx_ref, pad_ref):
    step = pl.program_id(0)

    @pl.when(step == 0)
    def _init():
        best_ref[...] = jnp.full((_V, 1), jnp.inf, jnp.float32)
        bidx_ref[...] = jnp.zeros((_V, 1), jnp.int32)

    v = ze_ref[...].reshape(_V, _D)
    for j in range(0, _KB, 2):
        diff0 = v - eblk_ref[j:j + 1, :]
        diff1 = v - eblk_ref[j + 1:j + 2, :]
        d0 = jnp.sum(diff0 * diff0, axis=1, keepdims=True)
        d1 = jnp.sum(diff1 * diff1, axis=1, keepdims=True)
        upd0 = d0 < best_ref[...]
        b0 = jnp.where(upd0, d0, best_ref[...])
        i0 = jnp.where(upd0, step * _KB + j, bidx_ref[...])
        upd1 = d1 < b0
        best_ref[...] = jnp.where(upd1, d1, b0)
        bidx_ref[...] = jnp.where(upd1, step * _KB + j + 1, i0)

    @pl.when(step == _STEPS - 1)
    def _finish():
        idx = bidx_ref[...]
        lane = jax.lax.broadcasted_iota(jnp.int32, (_V, _K), 1)
        onehot = (lane == idx).astype(jnp.float32)
        zq = jax.lax.dot_general(onehot, emb_ref[...], (((1,), (0,)), ((), ())),
                                 precision=jax.lax.Precision.HIGHEST,
                                 preferred_element_type=jnp.float32)
        zq_ref[...] = zq
        pad_ref[...] = jnp.zeros((_V, 2 * _W), jnp.float32)
        pad_ref[:, 0:_W] = zq
        wide = pad_ref[...]
        dec = jnp.zeros((_V, _W), jnp.float32)
        for t in range(_TK):
            shifted = pltpu.roll(wide, (_PAD - t) % (2 * _W), axis=1)[:, 0:_W]
            dec = dec + shifted * wcol_ref[t]
        for b in range(_B):
            blk = jnp.zeros((_H, _W), jnp.float32)
            for c in range(_C):
                blk = blk + dec[b * _C * _H + c * _H:b * _C * _H + c * _H + _H, :]
            xt_ref[b * _H:(b + 1) * _H, :] = jnp.tanh(blk)


def kernel(x, w_enc, bn_gamma, bn_beta, emb, w_dec):
    z = jax.lax.conv_general_dilated(
        x, w_enc, window_strides=(1, 1), padding=((0, 0), (_PAD, _PAD)),
        dimension_numbers=("NCHW", "OIHW", "NCHW"))
    mean = jnp.mean(z, axis=(0, 2, 3), keepdims=True)
    var = jnp.var(z, axis=(0, 2, 3), keepdims=True)
    z_e_x = (z - mean) / jnp.sqrt(var + _EPS)
    z_e_x = z_e_x * bn_gamma.reshape(1, -1, 1, 1) + bn_beta.reshape(1, -1, 1, 1)

    wt = w_dec[:, 0, 0, ::-1]
    c_idx = (jnp.arange(_V) // _H) % _C
    wcols = wt[c_idx].T[:, :, None]

    xt, zq = pl.pallas_call(
        _vq_kernel,
        grid=(_STEPS,),
        in_specs=[
            pl.BlockSpec((_B, _C, _H, _W), lambda s: (0, 0, 0, 0)),
            pl.BlockSpec((_KB, _D), lambda s: (s, 0)),
            pl.BlockSpec((_K, _D), lambda s: (0, 0)),
            pl.BlockSpec((_TK, _V, 1), lambda s: (0, 0, 0)),
        ],
        out_specs=(
            pl.BlockSpec((_B * _H, _W), lambda s: (0, 0)),
            pl.BlockSpec((_V, _D), lambda s: (0, 0)),
        ),
        out_shape=(
            jax.ShapeDtypeStruct((_B * _H, _W), jnp.float32),
            jax.ShapeDtypeStruct((_V, _D), jnp.float32),
        ),
        scratch_shapes=[
            pltpu.VMEM((_V, 1), jnp.float32),
            pltpu.VMEM((_V, 1), jnp.int32),
            pltpu.VMEM((_V, 2 * _W), jnp.float32),
        ],
    )(z_e_x, emb, emb, wcols)

    x_tilde = xt.reshape(_B, 1, _H, _W)
    z_q_x = zq.reshape(_B, _C, _H, _D)
    return (x_tilde, z_e_x, z_q_x)
